# baseline (speedup 1.0000x reference)
import functools

import jax
import jax.numpy as jnp
import numpy as np
from jax import lax
from jax.experimental import pallas as pl
from jax.experimental.pallas import tpu as pltpu
from jax.experimental.pallas import tpu_sc as plsc

TOP_K = 8
N_EXPERTS = 64
LANES = 16
SIGN = np.uint32(0x80000000)

_SORT8_NET = [(0, 1), (2, 3), (4, 5), (6, 7),
              (0, 2), (1, 3), (4, 6), (5, 7),
              (1, 2), (5, 6),
              (0, 4), (1, 5), (2, 6), (3, 7),
              (2, 4), (3, 5),
              (1, 2), (3, 4), (5, 6)]


def _tc_keys_body(x_ref, w_ref, out_ref):
    logits = lax.dot_general(
        w_ref[...], x_ref[...],
        dimension_numbers=(((1,), (1,)), ((), ())),
        precision=lax.Precision.DEFAULT,
        preferred_element_type=jnp.float32,
    )
    m = jnp.max(logits, axis=0, keepdims=True)
    e = jnp.exp(logits - m)
    s = jnp.sum(e, axis=0, keepdims=True)
    scores = e / s
    bits = lax.bitcast_convert_type(scores, jnp.uint32)
    inv_expert = (N_EXPERTS - 1) - lax.broadcasted_iota(
        jnp.uint32, scores.shape, 0)
    keys = lax.bitcast_convert_type(
        (((bits >> 4) << 6) | inv_expert) ^ SIGN, jnp.int32)
    rows = [keys[8 * r:8 * (r + 1)] for r in range(8)]
    for i, j in _SORT8_NET:
        a, b = rows[i], rows[j]
        rows[i] = jnp.maximum(a, b)
        rows[j] = jnp.minimum(a, b)
    sorted_keys = jnp.concatenate(rows, axis=0)
    spb = out_ref.shape[0]
    st = out_ref.shape[2]
    for j in range(spb):
        out_ref[j] = sorted_keys[:, j * st:(j + 1) * st]


def _tc_keys(x, weight, tblk, slab):
    n, h = x.shape
    grid = n // tblk
    spb = tblk // slab
    return pl.pallas_call(
        _tc_keys_body,
        grid=(grid,),
        in_specs=[
            pl.BlockSpec((tblk, h), lambda i: (i, 0)),
            pl.BlockSpec((N_EXPERTS, h), lambda i: (0, 0)),
        ],
        out_specs=pl.BlockSpec((spb, N_EXPERTS, slab), lambda i: (i, 0, 0)),
        out_shape=jax.ShapeDtypeStruct((n // slab, N_EXPERTS, slab),
                                       jnp.int32),
        compiler_params=pltpu.CompilerParams(
            dimension_semantics=("arbitrary",),
        ),
    )(x, weight)


def _sc_topk_kernel(n_tokens):
    info = plsc.get_sparse_core_info()
    nc, ns = info.num_cores, info.num_subcores
    nw = nc * ns
    tpw = n_tokens // nw
    groups = tpw // LANES
    mesh = plsc.VectorSubcoreMesh(core_axis_name="c", subcore_axis_name="s")

    @functools.partial(
        pl.kernel,
        mesh=mesh,
        out_type=(
            jax.ShapeDtypeStruct((TOP_K, n_tokens), jnp.int32),
            jax.ShapeDtypeStruct((TOP_K, n_tokens), jnp.int32),
        ),
        scratch_types=[
            pltpu.VMEM((N_EXPERTS, tpw), jnp.int32),
            pltpu.VMEM((TOP_K, tpw), jnp.int32),
            pltpu.VMEM((TOP_K, tpw), jnp.int32),
        ],
    )
    def body(keys_hbm, idx_hbm, wt_hbm, sv, ibuf, wbuf):
        wid = lax.axis_index("s") * nc + lax.axis_index("c")
        base = wid * tpw
        pltpu.sync_copy(keys_hbm.at[wid], sv)

        def ce(p, i, j):
            a, b = p[i], p[j]
            p[i] = jnp.maximum(a, b)
            p[j] = jnp.minimum(a, b)

        def merge8(a, b):
            w = [jnp.maximum(a[i], b[TOP_K - 1 - i]) for i in range(TOP_K)]
            for stride in (4, 2, 1):
                for bs in range(0, TOP_K, 2 * stride):
                    for off in range(stride):
                        ce(w, bs + off, bs + off + stride)
            return w

        def group_body(t, carry):
            toff = t * LANES

            def load_block(b):
                return [sv[b + 8 * r, pl.ds(toff, LANES)]
                        for r in range(TOP_K)]

            acc_a = load_block(0)
            acc_b = load_block(4)
            for s in range(1, 4):
                acc_a = merge8(acc_a, load_block(s))
                acc_b = merge8(acc_b, load_block(4 + s))
            top = merge8(acc_a, acc_b)
            sign_i = np.int32(-2**31)
            for k in range(TOP_K):
                key = top[k]
                widx = (N_EXPERTS - 1) - (key & (N_EXPERTS - 1))
                wt_bits = lax.shift_right_logical(key ^ sign_i, 6) << 4
                ibuf[k, pl.ds(toff, LANES)] = widx
                wbuf[k, pl.ds(toff, LANES)] = wt_bits
            return carry

        lax.fori_loop(0, groups, group_body, 0)

        pltpu.sync_copy(ibuf, idx_hbm.at[:, pl.ds(base, tpw)])
        pltpu.sync_copy(wbuf, wt_hbm.at[:, pl.ds(base, tpw)])

    return body


def kernel(hidden_states, weight):
    bsz, seqlen, hidden = hidden_states.shape
    n = bsz * seqlen
    x = hidden_states.reshape(n, hidden)
    keys_t = _tc_keys(x, weight, tblk=512, slab=n // 32)
    idx_t, wt_bits_t = _sc_topk_kernel(n)(keys_t)
    wt_t = lax.bitcast_convert_type(wt_bits_t, jnp.float32)
    topk_indices = idx_t.T.reshape(bsz, seqlen, TOP_K)
    topk_weights = wt_t.T.reshape(bsz, seqlen, TOP_K)
    return (topk_indices, topk_weights)


# speedup vs baseline: 1.1242x; 1.1242x over previous
import functools

import jax
import jax.numpy as jnp
import numpy as np
from jax import lax
from jax.experimental import pallas as pl
from jax.experimental.pallas import tpu as pltpu
from jax.experimental.pallas import tpu_sc as plsc

TOP_K = 8
N_EXPERTS = 64
LANES = 16
SIGN = np.uint32(0x80000000)

_SORT8_NET = [(0, 1), (2, 3), (4, 5), (6, 7),
              (0, 2), (1, 3), (4, 6), (5, 7),
              (1, 2), (5, 6),
              (0, 4), (1, 5), (2, 6), (3, 7),
              (2, 4), (3, 5),
              (1, 2), (3, 4), (5, 6)]


def _tc_keys_body(x_ref, w_ref, out_ref):
    logits = lax.dot_general(
        w_ref[...], x_ref[...],
        dimension_numbers=(((1,), (1,)), ((), ())),
        precision=lax.Precision.DEFAULT,
        preferred_element_type=jnp.float32,
    )
    m = jnp.max(logits, axis=0, keepdims=True)
    e = jnp.exp(logits - m)
    s = jnp.sum(e, axis=0, keepdims=True)
    scores = e / s
    bits = lax.bitcast_convert_type(scores, jnp.uint32)
    inv_expert = (N_EXPERTS - 1) - lax.broadcasted_iota(
        jnp.uint32, scores.shape, 0)
    keys = lax.bitcast_convert_type(
        (((bits >> 4) << 6) | inv_expert) ^ SIGN, jnp.int32)
    rows = [keys[8 * r:8 * (r + 1)] for r in range(8)]
    for i, j in _SORT8_NET:
        a, b = rows[i], rows[j]
        rows[i] = jnp.maximum(a, b)
        rows[j] = jnp.minimum(a, b)
    sorted_keys = jnp.concatenate(rows, axis=0)
    spb = out_ref.shape[0]
    st = out_ref.shape[2]
    for j in range(spb):
        out_ref[j] = sorted_keys[:, j * st:(j + 1) * st]


def _tc_keys(x, weight, tblk, slab):
    n, h = x.shape
    grid = n // tblk
    spb = tblk // slab
    return pl.pallas_call(
        _tc_keys_body,
        grid=(grid,),
        in_specs=[
            pl.BlockSpec((tblk, h), lambda i: (i, 0)),
            pl.BlockSpec((N_EXPERTS, h), lambda i: (0, 0)),
        ],
        out_specs=pl.BlockSpec((spb, N_EXPERTS, slab), lambda i: (i, 0, 0)),
        out_shape=jax.ShapeDtypeStruct((n // slab, N_EXPERTS, slab),
                                       jnp.int32),
        compiler_params=pltpu.CompilerParams(
            dimension_semantics=("arbitrary",),
        ),
    )(x, weight)


def _sc_topk_kernel(n_tokens):
    info = plsc.get_sparse_core_info()
    nc, ns = info.num_cores, info.num_subcores
    nw = nc * ns
    tpw = n_tokens // nw
    groups = tpw // LANES
    mesh = plsc.VectorSubcoreMesh(core_axis_name="c", subcore_axis_name="s")

    @functools.partial(
        pl.kernel,
        mesh=mesh,
        out_type=(
            jax.ShapeDtypeStruct((TOP_K, n_tokens), jnp.int32),
            jax.ShapeDtypeStruct((TOP_K, n_tokens), jnp.int32),
        ),
        scratch_types=[
            pltpu.VMEM((N_EXPERTS, tpw), jnp.int32),
            pltpu.VMEM((TOP_K, tpw), jnp.int32),
            pltpu.VMEM((TOP_K, tpw), jnp.int32),
        ],
    )
    def body(keys_hbm, idx_hbm, wt_hbm, sv, ibuf, wbuf):
        wid = lax.axis_index("s") * nc + lax.axis_index("c")
        base = wid * tpw
        pltpu.sync_copy(keys_hbm.at[wid], sv)

        def ce(p, i, j):
            a, b = p[i], p[j]
            p[i] = jnp.maximum(a, b)
            p[j] = jnp.minimum(a, b)

        def merge8(a, b):
            w = [jnp.maximum(a[i], b[TOP_K - 1 - i]) for i in range(TOP_K)]
            for stride in (4, 2, 1):
                for bs in range(0, TOP_K, 2 * stride):
                    for off in range(stride):
                        ce(w, bs + off, bs + off + stride)
            return w

        def group_body(t, carry):
            toff = t * LANES

            def load_block(b):
                return [sv[b + 8 * r, pl.ds(toff, LANES)]
                        for r in range(TOP_K)]

            acc_a = load_block(0)
            acc_b = load_block(4)
            for s in range(1, 4):
                acc_a = merge8(acc_a, load_block(s))
                acc_b = merge8(acc_b, load_block(4 + s))
            top = merge8(acc_a, acc_b)
            sign_i = np.int32(-2**31)
            for k in range(TOP_K):
                key = top[k]
                widx = (N_EXPERTS - 1) - (key & (N_EXPERTS - 1))
                wt_bits = lax.shift_right_logical(key ^ sign_i, 6) << 4
                ibuf[k, pl.ds(toff, LANES)] = widx
                wbuf[k, pl.ds(toff, LANES)] = wt_bits
            return carry

        lax.fori_loop(0, groups, group_body, 0)

        pltpu.sync_copy(ibuf, idx_hbm.at[:, pl.ds(base, tpw)])
        pltpu.sync_copy(wbuf, wt_hbm.at[:, pl.ds(base, tpw)])

    return body


def kernel(hidden_states, weight):
    bsz, seqlen, hidden = hidden_states.shape
    n = bsz * seqlen
    x = hidden_states.reshape(n, hidden)
    keys_t = _tc_keys(x, weight, tblk=1024, slab=n // 32)
    idx_t, wt_bits_t = _sc_topk_kernel(n)(keys_t)
    wt_t = lax.bitcast_convert_type(wt_bits_t, jnp.float32)
    topk_indices = idx_t.T.reshape(bsz, seqlen, TOP_K)
    topk_weights = wt_t.T.reshape(bsz, seqlen, TOP_K)
    return (topk_indices, topk_weights)


# speedup vs baseline: 1.1244x; 1.0002x over previous
import functools

import jax
import jax.numpy as jnp
import numpy as np
from jax import lax
from jax.experimental import pallas as pl
from jax.experimental.pallas import tpu as pltpu
from jax.experimental.pallas import tpu_sc as plsc

TOP_K = 8
N_EXPERTS = 64
LANES = 16
SIGN = np.uint32(0x80000000)

_SORT8_NET = [(0, 1), (2, 3), (4, 5), (6, 7),
              (0, 2), (1, 3), (4, 6), (5, 7),
              (1, 2), (5, 6),
              (0, 4), (1, 5), (2, 6), (3, 7),
              (2, 4), (3, 5),
              (1, 2), (3, 4), (5, 6)]


def _tc_keys_body(x_ref, w_ref, out_ref):
    logits = lax.dot_general(
        w_ref[...], x_ref[...],
        dimension_numbers=(((1,), (1,)), ((), ())),
        precision=lax.Precision.DEFAULT,
        preferred_element_type=jnp.float32,
    )
    m = jnp.max(logits, axis=0, keepdims=True)
    e = jnp.exp(logits - m)
    s = jnp.sum(e, axis=0, keepdims=True)
    scores = e / s
    bits = lax.bitcast_convert_type(scores, jnp.uint32)
    inv_expert = (N_EXPERTS - 1) - lax.broadcasted_iota(
        jnp.uint32, scores.shape, 0)
    keys = lax.bitcast_convert_type(
        (((bits >> 4) << 6) | inv_expert) ^ SIGN, jnp.int32)
    rows = [keys[8 * r:8 * (r + 1)] for r in range(8)]
    for i, j in _SORT8_NET:
        a, b = rows[i], rows[j]
        rows[i] = jnp.maximum(a, b)
        rows[j] = jnp.minimum(a, b)
    sorted_keys = jnp.concatenate(rows, axis=0)
    spb = out_ref.shape[0]
    st = out_ref.shape[2]
    for j in range(spb):
        out_ref[j] = sorted_keys[:, j * st:(j + 1) * st]


def _tc_keys(x, weight, tblk, slab):
    n, h = x.shape
    grid = n // tblk
    spb = tblk // slab
    return pl.pallas_call(
        _tc_keys_body,
        grid=(grid,),
        in_specs=[
            pl.BlockSpec((tblk, h), lambda i: (i, 0)),
            pl.BlockSpec((N_EXPERTS, h), lambda i: (0, 0)),
        ],
        out_specs=pl.BlockSpec((spb, N_EXPERTS, slab), lambda i: (i, 0, 0)),
        out_shape=jax.ShapeDtypeStruct((n // slab, N_EXPERTS, slab),
                                       jnp.int32),
        compiler_params=pltpu.CompilerParams(
            dimension_semantics=("arbitrary",),
        ),
    )(x, weight)


def _sc_topk_kernel(n_tokens):
    info = plsc.get_sparse_core_info()
    nc, ns = info.num_cores, info.num_subcores
    nw = nc * ns
    tpw = n_tokens // nw
    groups = tpw // LANES
    mesh = plsc.VectorSubcoreMesh(core_axis_name="c", subcore_axis_name="s")

    @functools.partial(
        pl.kernel,
        mesh=mesh,
        out_type=(
            jax.ShapeDtypeStruct((TOP_K, n_tokens), jnp.int32),
            jax.ShapeDtypeStruct((TOP_K, n_tokens), jnp.int32),
        ),
        scratch_types=[
            pltpu.VMEM((N_EXPERTS, tpw), jnp.int32),
            pltpu.VMEM((TOP_K, tpw), jnp.int32),
            pltpu.VMEM((TOP_K, tpw), jnp.int32),
            pltpu.SemaphoreType.DMA,
            pltpu.SemaphoreType.DMA,
        ],
    )
    def body(keys_hbm, idx_hbm, wt_hbm, sv, ibuf, wbuf, sem0, sem1):
        wid = lax.axis_index("s") * nc + lax.axis_index("c")
        base = wid * tpw
        half = tpw // 2
        cp0 = pltpu.async_copy(
            keys_hbm.at[wid, :, pl.ds(0, half)],
            sv.at[:, pl.ds(0, half)], sem0)
        cp1 = pltpu.async_copy(
            keys_hbm.at[wid, :, pl.ds(half, half)],
            sv.at[:, pl.ds(half, half)], sem1)

        def ce(p, i, j):
            a, b = p[i], p[j]
            p[i] = jnp.maximum(a, b)
            p[j] = jnp.minimum(a, b)

        def merge8(a, b):
            w = [jnp.maximum(a[i], b[TOP_K - 1 - i]) for i in range(TOP_K)]
            for stride in (4, 2, 1):
                for bs in range(0, TOP_K, 2 * stride):
                    for off in range(stride):
                        ce(w, bs + off, bs + off + stride)
            return w

        def group_body(t, carry):
            toff = t * LANES

            def load_block(b):
                return [sv[b + 8 * r, pl.ds(toff, LANES)]
                        for r in range(TOP_K)]

            acc_a = load_block(0)
            acc_b = load_block(4)
            for s in range(1, 4):
                acc_a = merge8(acc_a, load_block(s))
                acc_b = merge8(acc_b, load_block(4 + s))
            top = merge8(acc_a, acc_b)
            sign_i = np.int32(-2**31)
            for k in range(TOP_K):
                key = top[k]
                widx = (N_EXPERTS - 1) - (key & (N_EXPERTS - 1))
                wt_bits = lax.shift_right_logical(key ^ sign_i, 6) << 4
                ibuf[k, pl.ds(toff, LANES)] = widx
                wbuf[k, pl.ds(toff, LANES)] = wt_bits
            return carry

        cp0.wait()
        lax.fori_loop(0, groups // 2, group_body, 0)
        cp1.wait()
        lax.fori_loop(groups // 2, groups, group_body, 0)

        pltpu.sync_copy(ibuf, idx_hbm.at[:, pl.ds(base, tpw)])
        pltpu.sync_copy(wbuf, wt_hbm.at[:, pl.ds(base, tpw)])

    return body


def kernel(hidden_states, weight):
    bsz, seqlen, hidden = hidden_states.shape
    n = bsz * seqlen
    x = hidden_states.reshape(n, hidden)
    keys_t = _tc_keys(x, weight, tblk=1024, slab=n // 32)
    idx_t, wt_bits_t = _sc_topk_kernel(n)(keys_t)
    wt_t = lax.bitcast_convert_type(wt_bits_t, jnp.float32)
    topk_indices = idx_t.T.reshape(bsz, seqlen, TOP_K)
    topk_weights = wt_t.T.reshape(bsz, seqlen, TOP_K)
    return (topk_indices, topk_weights)


# speedup vs baseline: 1.1492x; 1.0221x over previous
import functools

import jax
import jax.numpy as jnp
import numpy as np
from jax import lax
from jax.experimental import pallas as pl
from jax.experimental.pallas import tpu as pltpu
from jax.experimental.pallas import tpu_sc as plsc

TOP_K = 8
N_EXPERTS = 64
LANES = 16
SIGN = np.uint32(0x80000000)

_SORT8_NET = [(0, 1), (2, 3), (4, 5), (6, 7),
              (0, 2), (1, 3), (4, 6), (5, 7),
              (1, 2), (5, 6),
              (0, 4), (1, 5), (2, 6), (3, 7),
              (2, 4), (3, 5),
              (1, 2), (3, 4), (5, 6)]


def _tc_keys_body(x_ref, w_ref, out_ref):
    logits = lax.dot_general(
        w_ref[...], x_ref[...],
        dimension_numbers=(((1,), (1,)), ((), ())),
        precision=lax.Precision.DEFAULT,
        preferred_element_type=jnp.float32,
    )
    m = jnp.max(logits, axis=0, keepdims=True)
    e = jnp.exp(logits - m)
    s = jnp.sum(e, axis=0, keepdims=True)
    scores = e / s
    bits = lax.bitcast_convert_type(scores, jnp.uint32)
    inv_expert = (N_EXPERTS - 1) - lax.broadcasted_iota(
        jnp.uint32, scores.shape, 0)
    keys = lax.bitcast_convert_type(
        (((bits >> 4) << 6) | inv_expert) ^ SIGN, jnp.int32)
    rows = [keys[8 * r:8 * (r + 1)] for r in range(8)]
    for i, j in _SORT8_NET:
        a, b = rows[i], rows[j]
        rows[i] = jnp.maximum(a, b)
        rows[j] = jnp.minimum(a, b)
    def rot4(v):
        return jnp.concatenate([v[4:], v[:4]], axis=0)

    w = [jnp.maximum(rows[i], rot4(rows[7 - i])) for i in range(8)]
    for stride in (4, 2, 1):
        for bs in range(0, 8, 2 * stride):
            for off in range(stride):
                i, j = bs + off, bs + off + stride
                a, b = w[i], w[j]
                w[i] = jnp.maximum(a, b)
                w[j] = jnp.minimum(a, b)
    half_keys = jnp.concatenate([w[i][0:4] for i in range(8)], axis=0)
    spb = out_ref.shape[0]
    st = out_ref.shape[2]
    for j in range(spb):
        out_ref[j] = half_keys[:, j * st:(j + 1) * st]


def _tc_keys(x, weight, tblk, slab):
    n, h = x.shape
    grid = n // tblk
    spb = tblk // slab
    return pl.pallas_call(
        _tc_keys_body,
        grid=(grid,),
        in_specs=[
            pl.BlockSpec((tblk, h), lambda i: (i, 0)),
            pl.BlockSpec((N_EXPERTS, h), lambda i: (0, 0)),
        ],
        out_specs=pl.BlockSpec((spb, 32, slab), lambda i: (i, 0, 0)),
        out_shape=jax.ShapeDtypeStruct((n // slab, 32, slab), jnp.int32),
        compiler_params=pltpu.CompilerParams(
            dimension_semantics=("arbitrary",),
        ),
    )(x, weight)


def _sc_topk_kernel(n_tokens):
    info = plsc.get_sparse_core_info()
    nc, ns = info.num_cores, info.num_subcores
    nw = nc * ns
    tpw = n_tokens // nw
    groups = tpw // LANES
    mesh = plsc.VectorSubcoreMesh(core_axis_name="c", subcore_axis_name="s")

    @functools.partial(
        pl.kernel,
        mesh=mesh,
        out_type=(
            jax.ShapeDtypeStruct((TOP_K, n_tokens), jnp.int32),
            jax.ShapeDtypeStruct((TOP_K, n_tokens), jnp.int32),
        ),
        scratch_types=[
            pltpu.VMEM((32, tpw), jnp.int32),
            pltpu.VMEM((TOP_K, tpw), jnp.int32),
            pltpu.VMEM((TOP_K, tpw), jnp.int32),
        ],
    )
    def body(keys_hbm, idx_hbm, wt_hbm, sv, ibuf, wbuf):
        wid = lax.axis_index("s") * nc + lax.axis_index("c")
        base = wid * tpw
        pltpu.sync_copy(keys_hbm.at[wid], sv)

        def ce(p, i, j):
            a, b = p[i], p[j]
            p[i] = jnp.maximum(a, b)
            p[j] = jnp.minimum(a, b)

        def merge8(a, b):
            w = [jnp.maximum(a[i], b[TOP_K - 1 - i]) for i in range(TOP_K)]
            for stride in (4, 2, 1):
                for bs in range(0, TOP_K, 2 * stride):
                    for off in range(stride):
                        ce(w, bs + off, bs + off + stride)
            return w

        def group_body(t, carry):
            toff = t * LANES

            def load_block(b):
                return [sv[b + 4 * r, pl.ds(toff, LANES)]
                        for r in range(TOP_K)]

            acc_a = merge8(load_block(0), load_block(1))
            acc_b = merge8(load_block(2), load_block(3))
            top = merge8(acc_a, acc_b)
            sign_i = np.int32(-2**31)
            for k in range(TOP_K):
                key = top[k]
                widx = (N_EXPERTS - 1) - (key & (N_EXPERTS - 1))
                wt_bits = lax.shift_right_logical(key ^ sign_i, 6) << 4
                ibuf[k, pl.ds(toff, LANES)] = widx
                wbuf[k, pl.ds(toff, LANES)] = wt_bits
            return carry

        lax.fori_loop(0, groups, group_body, 0)

        pltpu.sync_copy(ibuf, idx_hbm.at[:, pl.ds(base, tpw)])
        pltpu.sync_copy(wbuf, wt_hbm.at[:, pl.ds(base, tpw)])

    return body


def kernel(hidden_states, weight):
    bsz, seqlen, hidden = hidden_states.shape
    n = bsz * seqlen
    x = hidden_states.reshape(n, hidden)
    keys_t = _tc_keys(x, weight, tblk=1024, slab=n // 32)
    idx_t, wt_bits_t = _sc_topk_kernel(n)(keys_t)
    wt_t = lax.bitcast_convert_type(wt_bits_t, jnp.float32)
    topk_indices = idx_t.T.reshape(bsz, seqlen, TOP_K)
    topk_weights = wt_t.T.reshape(bsz, seqlen, TOP_K)
    return (topk_indices, topk_weights)


# speedup vs baseline: 1.1582x; 1.0078x over previous
import functools

import jax
import jax.numpy as jnp
import numpy as np
from jax import lax
from jax.experimental import pallas as pl
from jax.experimental.pallas import tpu as pltpu
from jax.experimental.pallas import tpu_sc as plsc

TOP_K = 8
N_EXPERTS = 64
LANES = 16
SIGN = np.uint32(0x80000000)

_SORT8_NET = [(0, 1), (2, 3), (4, 5), (6, 7),
              (0, 2), (1, 3), (4, 6), (5, 7),
              (1, 2), (5, 6),
              (0, 4), (1, 5), (2, 6), (3, 7),
              (2, 4), (3, 5),
              (1, 2), (3, 4), (5, 6)]


def _tc_keys_body(x_ref, w_ref, out_ref):
    logits = lax.dot_general(
        w_ref[...], x_ref[...],
        dimension_numbers=(((1,), (1,)), ((), ())),
        precision=lax.Precision.DEFAULT,
        preferred_element_type=jnp.float32,
    )
    m = jnp.max(logits, axis=0, keepdims=True)
    e = jnp.exp(logits - m)
    s = jnp.sum(e, axis=0, keepdims=True)
    scores = e / s
    bits = lax.bitcast_convert_type(scores, jnp.uint32)
    inv_expert = (N_EXPERTS - 1) - lax.broadcasted_iota(
        jnp.uint32, scores.shape, 0)
    keys = lax.bitcast_convert_type(
        (((bits >> 4) << 6) | inv_expert) ^ SIGN, jnp.int32)
    rows = [keys[8 * r:8 * (r + 1)] for r in range(8)]
    for i, j in _SORT8_NET:
        a, b = rows[i], rows[j]
        rows[i] = jnp.maximum(a, b)
        rows[j] = jnp.minimum(a, b)
    def rot(v, amt):
        return jnp.concatenate([v[amt:], v[:amt]], axis=0)

    def bitonic8_desc(w):
        for stride in (4, 2, 1):
            for bs in range(0, 8, 2 * stride):
                for off in range(stride):
                    i, j = bs + off, bs + off + stride
                    a, b = w[i], w[j]
                    w[i] = jnp.maximum(a, b)
                    w[j] = jnp.minimum(a, b)
        return w

    w = bitonic8_desc([jnp.maximum(rows[i], rot(rows[7 - i], 4))
                       for i in range(8)])
    w2 = bitonic8_desc([jnp.maximum(w[i], rot(w[7 - i], 2))
                        for i in range(8)])
    quarter_keys = jnp.concatenate([w2[i][0:2] for i in range(8)], axis=0)
    spb = out_ref.shape[0]
    st = out_ref.shape[2]
    for j in range(spb):
        out_ref[j] = quarter_keys[:, j * st:(j + 1) * st]


def _tc_keys(x, weight, tblk, slab):
    n, h = x.shape
    grid = n // tblk
    spb = tblk // slab
    return pl.pallas_call(
        _tc_keys_body,
        grid=(grid,),
        in_specs=[
            pl.BlockSpec((tblk, h), lambda i: (i, 0)),
            pl.BlockSpec((N_EXPERTS, h), lambda i: (0, 0)),
        ],
        out_specs=pl.BlockSpec((spb, 16, slab), lambda i: (i, 0, 0)),
        out_shape=jax.ShapeDtypeStruct((n // slab, 16, slab), jnp.int32),
        compiler_params=pltpu.CompilerParams(
            dimension_semantics=("arbitrary",),
        ),
    )(x, weight)


def _sc_topk_kernel(n_tokens):
    info = plsc.get_sparse_core_info()
    nc, ns = info.num_cores, info.num_subcores
    nw = nc * ns
    tpw = n_tokens // nw
    groups = tpw // LANES
    mesh = plsc.VectorSubcoreMesh(core_axis_name="c", subcore_axis_name="s")

    @functools.partial(
        pl.kernel,
        mesh=mesh,
        out_type=(
            jax.ShapeDtypeStruct((TOP_K, n_tokens), jnp.int32),
            jax.ShapeDtypeStruct((TOP_K, n_tokens), jnp.int32),
        ),
        scratch_types=[
            pltpu.VMEM((16, tpw), jnp.int32),
            pltpu.VMEM((TOP_K, tpw), jnp.int32),
            pltpu.VMEM((TOP_K, tpw), jnp.int32),
        ],
    )
    def body(keys_hbm, idx_hbm, wt_hbm, sv, ibuf, wbuf):
        wid = lax.axis_index("s") * nc + lax.axis_index("c")
        base = wid * tpw
        pltpu.sync_copy(keys_hbm.at[wid], sv)

        def ce(p, i, j):
            a, b = p[i], p[j]
            p[i] = jnp.maximum(a, b)
            p[j] = jnp.minimum(a, b)

        def merge8(a, b):
            w = [jnp.maximum(a[i], b[TOP_K - 1 - i]) for i in range(TOP_K)]
            for stride in (4, 2, 1):
                for bs in range(0, TOP_K, 2 * stride):
                    for off in range(stride):
                        ce(w, bs + off, bs + off + stride)
            return w

        def group_body(t, carry):
            toff = t * LANES

            def load_block(b):
                return [sv[b + 2 * r, pl.ds(toff, LANES)]
                        for r in range(TOP_K)]

            top = merge8(load_block(0), load_block(1))
            sign_i = np.int32(-2**31)
            for k in range(TOP_K):
                key = top[k]
                widx = (N_EXPERTS - 1) - (key & (N_EXPERTS - 1))
                wt_bits = lax.shift_right_logical(key ^ sign_i, 6) << 4
                ibuf[k, pl.ds(toff, LANES)] = widx
                wbuf[k, pl.ds(toff, LANES)] = wt_bits
            return carry

        lax.fori_loop(0, groups, group_body, 0)

        pltpu.sync_copy(ibuf, idx_hbm.at[:, pl.ds(base, tpw)])
        pltpu.sync_copy(wbuf, wt_hbm.at[:, pl.ds(base, tpw)])

    return body


def kernel(hidden_states, weight):
    bsz, seqlen, hidden = hidden_states.shape
    n = bsz * seqlen
    x = hidden_states.reshape(n, hidden)
    keys_t = _tc_keys(x, weight, tblk=1024, slab=n // 32)
    idx_t, wt_bits_t = _sc_topk_kernel(n)(keys_t)
    wt_t = lax.bitcast_convert_type(wt_bits_t, jnp.float32)
    topk_indices = idx_t.T.reshape(bsz, seqlen, TOP_K)
    topk_weights = wt_t.T.reshape(bsz, seqlen, TOP_K)
    return (topk_indices, topk_weights)
